# baseline (speedup 1.0000x reference)
import functools

import jax
import jax.numpy as jnp
from jax import lax
from jax.experimental import pallas as pl
from jax.experimental.pallas import tpu as pltpu
from jax.experimental.pallas import tpu_sc as plsc

DIM = 128
B = 4096
L = 20
NW = 32
BW = B // NW
KW = BW * L
SPC = 4
CHUNK = SPC * L
NCHUNK = KW // CHUNK
NBLK = CHUNK // 16
SENTINEL = -1e30

_MESH = plsc.VectorSubcoreMesh(core_axis_name="c", subcore_axis_name="s")

_GATHER_DNUMS = lax.GatherDimensionNumbers(
    offset_dims=(), collapsed_slice_dims=(0,), start_index_map=(0,))


def _shfl(v, idx):
    return lax.gather(v, idx[:, None], _GATHER_DNUMS, slice_sizes=(1,),
                      mode=lax.GatherScatterMode.PROMISE_IN_BOUNDS)


@functools.partial(
    pl.kernel,
    out_type=jax.ShapeDtypeStruct((NW, KW), jnp.float32),
    mesh=_MESH,
    scratch_types=[
        pltpu.VMEM((KW,), jnp.int32),
        pltpu.VMEM((2 * CHUNK, DIM), jnp.float32),
        pltpu.VMEM((BW, DIM), jnp.float32),
        pltpu.VMEM((KW,), jnp.int32),
        pltpu.VMEM((BW + 16,), jnp.int32),
        pltpu.VMEM((KW,), jnp.float32),
        pltpu.SemaphoreType.DMA,
        pltpu.SemaphoreType.DMA,
    ],
)
def _sc_logits(table_hbm, tp_hbm, hid_hbm, code_hbm, len_hbm, out_hbm,
               idx_v, rows_v, hid_v, code_v, len_v, z_v, sem0, sem1):
    wid = lax.axis_index("c") * 16 + lax.axis_index("s")
    base = wid * BW
    pltpu.sync_copy(tp_hbm.at[wid], idx_v)
    pltpu.sync_copy(hid_hbm.at[pl.ds(base, BW)], hid_v)
    pltpu.sync_copy(code_hbm.at[wid], code_v)
    pltpu.sync_copy(len_hbm.at[pl.ds(base, BW)], len_v.at[pl.ds(0, BW)])
    lanes = lax.iota(jnp.int32, 16)

    half0, half1 = rows_v.at[pl.ds(0, CHUNK)], rows_v.at[pl.ds(CHUNK, CHUNK)]

    def start_gather(c, buf, sem):
        cc = jnp.minimum(c, NCHUNK - 1)
        pltpu.async_copy(table_hbm.at[idx_v.at[pl.ds(cc * CHUNK, CHUNK)]], buf, sem)

    def wait_gather(buf, sem):
        pltpu.make_async_copy(
            table_hbm.at[idx_v.at[pl.ds(0, CHUNK)]], buf, sem).wait()

    def compute(c, off):
        kbase = c * CHUNK
        hs = []
        lns = []
        for bb in range(SPC):
            sb = c * SPC + bb
            hs.append([hid_v[sb, pl.ds(s * 16, 16)] for s in range(DIM // 16)])
            lns.append(len_v[pl.ds(sb, 16)][0])
        zvecs = [jnp.zeros((16,), jnp.float32)] * NBLK
        for r in range(CHUNK):
            h = hs[r // L]
            acc = None
            for s in range(DIM // 16):
                prod = rows_v[off + r, pl.ds(s * 16, 16)] * h[s]
                acc = prod if acc is None else acc + prod
            for sh in (8, 4, 2, 1):
                acc = acc + _shfl(acc, jnp.bitwise_xor(lanes, sh))
            t, u = r // 16, r % 16
            zvecs[t] = jnp.where(lanes == u, acc, zvecs[t])
        for t in range(NBLK):
            r0 = t * 16
            bb0, bb1 = r0 // L, (r0 + 15) // L
            ub = bb1 * L - r0
            lconst = (r0 % L) + lanes
            lconst = jnp.where(lconst >= L, lconst - L, lconst)
            if bb0 == bb1:
                lenvec = jnp.broadcast_to(lns[bb0], (16,))
            else:
                lenvec = jnp.where(lanes < ub, lns[bb0], lns[bb1])
            codev = code_v[pl.ds(kbase + r0, 16)]
            sign = 1.0 - 2.0 * codev.astype(jnp.float32)
            x = jnp.where(lconst < lenvec, zvecs[t] * sign,
                          jnp.float32(SENTINEL))
            z_v[pl.ds(kbase + r0, 16)] = x

    start_gather(0, half0, sem0)

    def chunk_body(c, _):
        even = c % 2 == 0

        @pl.when(even)
        def _():
            wait_gather(half0, sem0)
            start_gather(c + 1, half1, sem1)

        @pl.when(jnp.logical_not(even))
        def _():
            wait_gather(half1, sem1)
            start_gather(c + 1, half0, sem0)

        compute(c, (c % 2) * CHUNK)
        return 0

    lax.fori_loop(0, NCHUNK, chunk_body, 0)
    wait_gather(half0, sem0)
    pltpu.sync_copy(z_v, out_hbm.at[wid])


def _tc_loss_body(x_ref, out_ref):
    x = x_ref[...]
    loss = jnp.maximum(x, 0.0) + jnp.log(1.0 + jnp.exp(-jnp.abs(x)))
    cnt = jnp.sum((x > SENTINEL * 0.5).astype(jnp.float32))
    out_ref[...] = (jnp.sum(loss) / cnt).reshape(1, 1)


_tc_loss = pl.pallas_call(
    _tc_loss_body,
    out_shape=jax.ShapeDtypeStruct((1, 1), jnp.float32),
)


def kernel(hidden_, target, target_path, target_path_len, target_code, embed_table):
    tp = target_path.reshape(NW, KW)
    code = target_code.reshape(NW, KW)
    x = _sc_logits(embed_table, tp, hidden_, code, target_path_len)
    loss = _tc_loss(x)
    return loss[0, 0]
